# baseline (speedup 1.0000x reference)
import jax
import jax.numpy as jnp
import numpy as np
from jax.experimental import pallas as pl

B = 256
L = 512
AA = 20
CH = 8
TB = 32
NR = AA * TB


def _pm_const():
    rng = np.random.RandomState(0)
    m = rng.randn(AA, AA).astype(np.float32)
    m = (m + m.T) / 2.0
    return np.clip(m, 0.0, None)


def _extract_mat():
    rows = [[(sid, 0)] for sid in range(NR)]
    for t in range(1, 10):
        if len(rows) == 1:
            break
        delta = 1 << (t - 1)
        r = len(rows)
        h2 = (r + 1) // 2
        rows = [rows[j] + ([(s, p + delta) for (s, p) in rows[j + h2]]
                           if j + h2 < r else [])
                for j in range(h2)]
    e = np.zeros((len(rows), L, NR), np.float32)
    for r, ents in enumerate(rows):
        for sid, ph in ents:
            e[r, ph, sid] = 1.0
    return e


def _shift_r(a, s):
    return jnp.concatenate(
        [jnp.zeros_like(a[..., :s]), a[..., : a.shape[-1] - s]], axis=-1)


def _shift_l(a, s):
    return jnp.concatenate(
        [a[..., s:], jnp.zeros_like(a[..., :s])], axis=-1)


def _merge(zs, t):
    r = zs[0].shape[0]
    h2 = (r + 1) // 2
    delta = 1 << (t - 1)
    li = jax.lax.broadcasted_iota(jnp.int32, (h2, L), 1)
    keep = (li % (1 << t)) < delta
    out = []
    for z in zs:
        lo = z[:h2]
        up = z[h2:]
        if up.shape[0] < h2:
            up = jnp.concatenate(
                [up, jnp.zeros((h2 - up.shape[0], L), jnp.float32)], axis=0)
        out.append(jnp.where(keep, lo, _shift_r(up, delta)))
    return out


def _fused_kernel(xp_ref, v_ref, vk_ref, e_ref, w0_ref, *rest):
    w_refs = rest[:-1]
    out_ref = rest[-1]

    xp = xp_ref[...]
    nzf = jnp.where(xp != 0.0, 1.0, 0.0)
    vmat = v_ref[...]

    aidx = jax.lax.broadcasted_iota(jnp.int32, (AA, 1, 1), 0)
    col = jnp.zeros_like(xp)
    for i in range(AA):
        dm = jnp.where(aidx != i, 1.0, 0.0)
        wm = nzf[i : i + 1, :, :] * dm
        col = jnp.where(wm > 0.5, vmat[i].reshape(AA, 1, 1), col)

    li3 = jax.lax.broadcasted_iota(jnp.int32, (AA, TB, L), 2)
    row = jnp.zeros_like(xp)
    for k in (1, 2, 3):
        vk = vk_ref[:, :, k - 1 : k]
        sr = _shift_r(nzf, k)
        suf = jnp.max(nzf[..., L - 1 - k :], axis=-1, keepdims=True)
        mp = jnp.where(li3 == L - 1, suf, sr)
        row = jnp.where(mp > 0.5, vk, row)
        sl = _shift_l(nzf, k)
        pre = jnp.max(nzf[..., : k + 1], axis=-1, keepdims=True)
        mm = jnp.where(li3 == 0, pre, sl)
        row = jnp.where(mm > 0.5, vk, row)

    y = (xp + col + row).reshape(NR, L)

    rnd = lambda a: a.astype(jnp.bfloat16).astype(jnp.float32)

    w0 = rnd(w0_ref[...])
    y = rnd(y)
    yr = _shift_r(y, 1)
    yl = _shift_l(y, 1)
    zc = []
    for o in range(CH):
        c = (w0[o : o + 1, 0, 0:1] * yr
             + w0[o : o + 1, 0, 1:2] * y
             + w0[o : o + 1, 0, 2:3] * yl)
        zc.append(c + _shift_l(c, 1))
    zc = _merge(zc, 1)

    s = 2
    for t, w_ref in enumerate(w_refs, start=2):
        w = rnd(w_ref[...])
        zc = [rnd(z) for z in zc]
        rs = [_shift_r(z, s) for z in zc]
        ls = [_shift_l(z, s) for z in zc]
        nzc = []
        for o in range(CH):
            acc = None
            for i in range(CH):
                term = (w[o : o + 1, i, 0:1] * rs[i]
                        + w[o : o + 1, i, 1:2] * zc[i]
                        + w[o : o + 1, i, 2:3] * ls[i])
                acc = term if acc is None else acc + term
            nzc.append(acc + _shift_l(acc, s))
        if nzc[0].shape[0] > 1:
            nzc = _merge(nzc, t)
        zc = nzc
        s *= 2

    nfr = zc[0].shape[0]
    res = None
    for r in range(nfr):
        z8 = jnp.concatenate([z[r : r + 1] for z in zc], axis=0)
        part = jax.lax.dot_general(
            z8, e_ref[r], (((1,), (0,)), ((), ())),
            precision=jax.lax.Precision.HIGHEST,
            preferred_element_type=jnp.float32)
        res = part if res is None else res + part
    out_ref[...] = res[None]


@jax.jit
def kernel(x, masks, lpm, std, W0, W1, W2, W3, W4, W5, W6, W7, W8):
    del masks
    pm = jnp.asarray(_pm_const())
    u = jnp.clip(lpm, 0.001, 1.0) * pm
    ii = jnp.arange(AA)[:, None]
    dd = jnp.arange(AA)[None, :]
    vmat = jnp.where(ii < dd, u, u.T)
    vmat = vmat.at[:, AA - 1].set(0.0)

    ks = jnp.arange(1, 4, dtype=jnp.float32)
    vks = jnp.exp(-(ks * ks) / (2.0 * std * std)).reshape(1, 1, 3)

    emat = jnp.asarray(0.5 * _extract_mat())
    ws = [0.5 * w for w in (W1, W2, W3, W4, W5, W6, W7, W8)]
    xp3 = jnp.transpose(x, (2, 0, 1))

    g = B // TB
    full = lambda shape: pl.BlockSpec(shape, lambda i: (0,) * len(shape))
    in_specs = [
        pl.BlockSpec((AA, TB, L), lambda i: (0, i, 0)),
        full((AA, AA)),
        full((1, 1, 3)),
        full(( _extract_mat().shape[0], L, NR)),
        full((CH, 1, 3)),
    ] + [full((CH, CH, 3))] * 8
    out_spec = pl.BlockSpec((1, CH, NR), lambda i: (i, 0, 0))

    out = pl.pallas_call(
        _fused_kernel,
        grid=(g,),
        in_specs=in_specs,
        out_specs=out_spec,
        out_shape=jax.ShapeDtypeStruct((g, CH, NR), jnp.float32),
    )(xp3, vmat, vks, emat, W0, *ws)

    res = out.reshape(g, CH, AA, TB)
    res = jnp.transpose(res, (0, 3, 2, 1))
    return res.reshape(B, AA, CH)


# speedup vs baseline: 1.1944x; 1.1944x over previous
import jax
import jax.numpy as jnp
import numpy as np
from jax.experimental import pallas as pl

B = 256
L = 512
AA = 20
CH = 8
TB = 16
NR = AA * TB


def _pm_const():
    rng = np.random.RandomState(0)
    m = rng.randn(AA, AA).astype(np.float32)
    m = (m + m.T) / 2.0
    return np.clip(m, 0.0, None)


def _extract_mat():
    rows = [[(sid, 0)] for sid in range(NR)]
    for t in range(1, 10):
        if len(rows) == 1:
            break
        delta = 1 << (t - 1)
        r = len(rows)
        h2 = (r + 1) // 2
        rows = [rows[j] + ([(s, p + delta) for (s, p) in rows[j + h2]]
                           if j + h2 < r else [])
                for j in range(h2)]
    e = np.zeros((len(rows), L, NR), np.float32)
    for r, ents in enumerate(rows):
        for sid, ph in ents:
            e[r, ph, sid] = 1.0
    return e


def _shift_r(a, s):
    return jnp.concatenate(
        [jnp.zeros_like(a[..., :s]), a[..., : a.shape[-1] - s]], axis=-1)


def _shift_l(a, s):
    return jnp.concatenate(
        [a[..., s:], jnp.zeros_like(a[..., :s])], axis=-1)


def _merge(zs, t):
    r = zs[0].shape[0]
    h2 = (r + 1) // 2
    delta = 1 << (t - 1)
    li = jax.lax.broadcasted_iota(jnp.int32, (h2, L), 1)
    keep = (li % (1 << t)) < delta
    out = []
    for z in zs:
        lo = z[:h2]
        up = z[h2:]
        if up.shape[0] < h2:
            up = jnp.concatenate(
                [up, jnp.zeros((h2 - up.shape[0], L), jnp.float32)], axis=0)
        out.append(jnp.where(keep, lo, _shift_r(up, delta)))
    return out


def _fused_kernel(xp_ref, v_ref, vk_ref, e_ref, w0_ref, *rest):
    w_refs = rest[:-1]
    out_ref = rest[-1]

    xp = xp_ref[...]
    nzf = jnp.where(xp != 0.0, 1.0, 0.0)
    vmat = v_ref[...]

    aidx = jax.lax.broadcasted_iota(jnp.int32, (AA, 1, 1), 0)
    col = jnp.zeros_like(xp)
    for i in range(AA):
        dm = jnp.where(aidx != i, 1.0, 0.0)
        wm = nzf[i : i + 1, :, :] * dm
        col = jnp.where(wm > 0.5, vmat[i].reshape(AA, 1, 1), col)

    li3 = jax.lax.broadcasted_iota(jnp.int32, (AA, TB, L), 2)
    row = jnp.zeros_like(xp)
    for k in (1, 2, 3):
        vk = vk_ref[:, :, k - 1 : k]
        sr = _shift_r(nzf, k)
        suf = jnp.max(nzf[..., L - 1 - k :], axis=-1, keepdims=True)
        mp = jnp.where(li3 == L - 1, suf, sr)
        row = jnp.where(mp > 0.5, vk, row)
        sl = _shift_l(nzf, k)
        pre = jnp.max(nzf[..., : k + 1], axis=-1, keepdims=True)
        mm = jnp.where(li3 == 0, pre, sl)
        row = jnp.where(mm > 0.5, vk, row)

    y = (xp + col + row).reshape(NR, L)

    rnd = lambda a: a.astype(jnp.bfloat16).astype(jnp.float32)

    w0 = rnd(w0_ref[...])
    py = rnd(y)
    py = py + _shift_l(py, 1)
    yr = _shift_r(py, 1)
    yl = _shift_l(py, 1)
    zc = []
    for o in range(CH):
        c = (w0[o : o + 1, 0, 0:1] * yr
             + w0[o : o + 1, 0, 1:2] * py
             + w0[o : o + 1, 0, 2:3] * yl)
        zc.append(c)
    zc = _merge(zc, 1)

    s = 2
    for t, w_ref in enumerate(w_refs, start=2):
        w = rnd(w_ref[...])
        zc = [rnd(z) for z in zc]
        rs = [_shift_r(z, s) for z in zc]
        ls = [_shift_l(z, s) for z in zc]
        nzc = []
        for o in range(CH):
            acc = None
            for i in range(CH):
                term = (w[o : o + 1, i, 0:1] * rs[i]
                        + w[o : o + 1, i, 1:2] * zc[i]
                        + w[o : o + 1, i, 2:3] * ls[i])
                acc = term if acc is None else acc + term
            nzc.append(acc + _shift_l(acc, s))
        if nzc[0].shape[0] > 1:
            nzc = _merge(nzc, t)
        zc = nzc
        s *= 2

    nfr = zc[0].shape[0]
    res = None
    for r in range(nfr):
        z8 = jnp.concatenate([z[r : r + 1] for z in zc], axis=0)
        part = jax.lax.dot_general(
            z8, e_ref[r], (((1,), (0,)), ((), ())),
            precision=jax.lax.Precision.HIGHEST,
            preferred_element_type=jnp.float32)
        res = part if res is None else res + part
    out_ref[...] = res[None]


@jax.jit
def kernel(x, masks, lpm, std, W0, W1, W2, W3, W4, W5, W6, W7, W8):
    del masks
    pm = jnp.asarray(_pm_const())
    u = jnp.clip(lpm, 0.001, 1.0) * pm
    ii = jnp.arange(AA)[:, None]
    dd = jnp.arange(AA)[None, :]
    vmat = jnp.where(ii < dd, u, u.T)
    vmat = vmat.at[:, AA - 1].set(0.0)

    ks = jnp.arange(1, 4, dtype=jnp.float32)
    vks = jnp.exp(-(ks * ks) / (2.0 * std * std)).reshape(1, 1, 3)

    emat = jnp.asarray(0.5 * _extract_mat())
    ws = [0.5 * w for w in (W1, W2, W3, W4, W5, W6, W7, W8)]
    xp3 = jnp.transpose(x, (2, 0, 1))

    g = B // TB
    full = lambda shape: pl.BlockSpec(shape, lambda i: (0,) * len(shape))
    in_specs = [
        pl.BlockSpec((AA, TB, L), lambda i: (0, i, 0)),
        full((AA, AA)),
        full((1, 1, 3)),
        full(( _extract_mat().shape[0], L, NR)),
        full((CH, 1, 3)),
    ] + [full((CH, CH, 3))] * 8
    out_spec = pl.BlockSpec((1, CH, NR), lambda i: (i, 0, 0))

    out = pl.pallas_call(
        _fused_kernel,
        grid=(g,),
        in_specs=in_specs,
        out_specs=out_spec,
        out_shape=jax.ShapeDtypeStruct((g, CH, NR), jnp.float32),
    )(xp3, vmat, vks, emat, W0, *ws)

    res = out.reshape(g, CH, AA, TB)
    res = jnp.transpose(res, (0, 3, 2, 1))
    return res.reshape(B, AA, CH)
